# baseline (speedup 1.0000x reference)
import jax
import jax.numpy as jnp
from jax import lax
from jax.experimental import pallas as pl
from jax.experimental.pallas import tpu as pltpu
from jax.experimental.pallas import tpu_sc as plsc

MAXLEN = 200
EMB_DIM = 64
SMALL_DIM = 32
OUT_DIM = EMB_DIM + 4 * SMALL_DIM

B, L = 4096, 200
N_TOK = B * L

POS_OFF = MAXLEN
NER_OFF = MAXLEN + 48
SMALL_ROWS = MAXLEN + 48 + 20

NUM_CORES = 2
NUM_SUBCORES = 16
NUM_WORKERS = NUM_CORES * NUM_SUBCORES
SEQ_PER_WORKER = B // NUM_WORKERS
N_PAIRS = SEQ_PER_WORKER // 2
LANES = 16
SEG = 208


def _body(words, subj, obj, pos, ner, word_table, small_table, out, *scratch):
  (wi_a, si_a, wr_a, sr_a, wi_b, si_b, wr_b, sr_b,
   semg_a, sems_a, semg_b, sems_b) = scratch
  slot_a = (wi_a, si_a, wr_a, sr_a, semg_a, sems_a)
  slot_b = (wi_b, si_b, wr_b, sr_b, semg_b, sems_b)

  c = lax.axis_index("c")
  s = lax.axis_index("s")
  wid = s * NUM_CORES + c
  rep_base = wid * SMALL_ROWS

  def start(i, slot):
    wi, si, wr, sr, semg, _ = slot
    seq = wid * SEQ_PER_WORKER + i
    tok0 = seq * L
    pltpu.sync_copy(words.at[pl.ds(tok0, L)], wi)
    pltpu.sync_copy(subj.at[pl.ds(tok0, SEG)], si.at[pl.ds(0, SEG)])
    pltpu.sync_copy(obj.at[pl.ds(tok0, SEG)], si.at[pl.ds(SEG, SEG)])
    pltpu.sync_copy(pos.at[pl.ds(tok0, SEG)], si.at[pl.ds(2 * SEG, SEG)])
    pltpu.sync_copy(ner.at[pl.ds(tok0, SEG)], si.at[pl.ds(3 * SEG, SEG)])
    for k in range(2 * SEG // LANES):
      sl = pl.ds(k * LANES, LANES)
      si[sl] = si[sl] + rep_base
    for k in range(SEG // LANES):
      sl = pl.ds(2 * SEG + k * LANES, LANES)
      si[sl] = si[sl] + (rep_base + POS_OFF)
    for k in range(SEG // LANES):
      sl = pl.ds(3 * SEG + k * LANES, LANES)
      si[sl] = si[sl] + (rep_base + NER_OFF)
    pltpu.async_copy(word_table.at[wi], wr, semg)
    pltpu.async_copy(small_table.at[si], sr, semg)

  def wait_gathers(slot):
    wi, si, wr, sr, semg, _ = slot
    pltpu.make_async_copy(word_table.at[wi], wr, semg).wait()
    pltpu.make_async_copy(small_table.at[si], sr, semg).wait()

  def scatter_ops(i, slot):
    _, _, wr, sr, _, sems = slot
    seq = wid * SEQ_PER_WORKER + i
    ops = [(wr, out.at[seq, :, pl.ds(0, EMB_DIM)], sems)]
    for f in range(4):
      ops.append((sr.at[pl.ds(f * SEG, L)],
                  out.at[seq, :, pl.ds(EMB_DIM + f * SMALL_DIM, SMALL_DIM)],
                  sems))
    return ops

  def fire_scatters(i, slot):
    for src, dst, sem in scatter_ops(i, slot):
      pltpu.async_copy(src, dst, sem)

  def drain_scatters(i, slot):
    for src, dst, sem in scatter_ops(i, slot):
      pltpu.make_async_copy(src, dst, sem).wait()

  start(0, slot_a)

  @pl.loop(0, N_PAIRS)
  def _pair(j):
    i0 = 2 * j
    i1 = i0 + 1

    @pl.when(j > 0)
    def _():
      drain_scatters(i0, slot_b)
    start(i1, slot_b)
    wait_gathers(slot_a)
    fire_scatters(i0, slot_a)

    @pl.when(j < N_PAIRS - 1)
    def _():
      drain_scatters(i0, slot_a)
      start(i0 + 2, slot_a)
    wait_gathers(slot_b)
    fire_scatters(i1, slot_b)

  drain_scatters(0, slot_a)
  drain_scatters(0, slot_b)


@jax.jit
def _run(words, subj, obj, pos, ner, word_table, small_table):
  mesh = plsc.VectorSubcoreMesh(
      core_axis_name="c", subcore_axis_name="s",
      num_cores=NUM_CORES, num_subcores=NUM_SUBCORES)
  grid_kernel = pl.kernel(
      _body,
      out_type=jax.ShapeDtypeStruct((B, L, OUT_DIM), jnp.float32),
      mesh=mesh,
      scratch_types=[
          pltpu.VMEM((L,), jnp.int32),
          pltpu.VMEM((4 * SEG,), jnp.int32),
          pltpu.VMEM((L, EMB_DIM), jnp.float32),
          pltpu.VMEM((4 * SEG, SMALL_DIM), jnp.float32),
          pltpu.VMEM((L,), jnp.int32),
          pltpu.VMEM((4 * SEG,), jnp.int32),
          pltpu.VMEM((L, EMB_DIM), jnp.float32),
          pltpu.VMEM((4 * SEG, SMALL_DIM), jnp.float32),
          pltpu.SemaphoreType.DMA,
          pltpu.SemaphoreType.DMA,
          pltpu.SemaphoreType.DMA,
          pltpu.SemaphoreType.DMA,
      ],
      compiler_params=pltpu.CompilerParams(use_tc_tiling_on_sc=False),
      name="embed_concat_sc",
  )
  return grid_kernel(words, subj, obj, pos, ner, word_table, small_table)


def kernel(words, pos, ner, subj_pos, obj_pos,
           word_table, pos_table, ner_table, position_table):
  small_table = jnp.tile(
      jnp.concatenate([position_table[MAXLEN:], pos_table, ner_table], axis=0),
      (NUM_WORKERS, 1))
  pad = SEG - L
  return _run(words.reshape(N_TOK),
              jnp.pad(subj_pos.reshape(N_TOK), (0, pad)),
              jnp.pad(obj_pos.reshape(N_TOK), (0, pad)),
              jnp.pad(pos.reshape(N_TOK), (0, pad)),
              jnp.pad(ner.reshape(N_TOK), (0, pad)),
              word_table, small_table)


# speedup vs baseline: 1.0460x; 1.0460x over previous
import jax
import jax.numpy as jnp
from jax import lax
from jax.experimental import pallas as pl
from jax.experimental.pallas import tpu as pltpu
from jax.experimental.pallas import tpu_sc as plsc

MAXLEN = 200
EMB_DIM = 64
SMALL_DIM = 32
OUT_DIM = EMB_DIM + 4 * SMALL_DIM

B, L = 4096, 200
N_TOK = B * L

POS_OFF = MAXLEN
NER_OFF = MAXLEN + 48
SMALL_ROWS = MAXLEN + 48 + 20

NUM_CORES = 2
NUM_SUBCORES = 16
NUM_WORKERS = NUM_CORES * NUM_SUBCORES
TOK_PER_WORKER = N_TOK // NUM_WORKERS
CHUNK = 320
N_CHUNKS = TOK_PER_WORKER // CHUNK
N_PAIRS = N_CHUNKS // 2
LANES = 16


def _body(words, subj, obj, pos, ner, word_table, small_table, out, *scratch):
  (wi_a, si_a, wr_a, sr_a, wi_b, si_b, wr_b, sr_b,
   semi_a, semg_a, sems_a, semi_b, semg_b, sems_b) = scratch
  slot_a = (wi_a, si_a, wr_a, sr_a, semi_a, semg_a, sems_a)
  slot_b = (wi_b, si_b, wr_b, sr_b, semi_b, semg_b, sems_b)

  c = lax.axis_index("c")
  s = lax.axis_index("s")
  wid = s * NUM_CORES + c
  rep_base = wid * SMALL_ROWS

  def idx_ops(i, slot):
    wi, si = slot[0], slot[1]
    semi = slot[4]
    base = wid * TOK_PER_WORKER + i * CHUNK
    tok = pl.ds(base, CHUNK)
    return [
        (words.at[tok], wi, semi),
        (subj.at[tok], si.at[pl.ds(0, CHUNK)], semi),
        (obj.at[tok], si.at[pl.ds(CHUNK, CHUNK)], semi),
        (pos.at[tok], si.at[pl.ds(2 * CHUNK, CHUNK)], semi),
        (ner.at[tok], si.at[pl.ds(3 * CHUNK, CHUNK)], semi),
    ]

  def start(i, slot):
    wi, si, wr, sr, semi, semg, _ = slot
    ops = idx_ops(i, slot)
    for src, dst, sem in ops:
      pltpu.async_copy(src, dst, sem)
    for src, dst, sem in ops:
      pltpu.make_async_copy(src, dst, sem).wait()
    for k in range(2 * CHUNK // LANES):
      sl = pl.ds(k * LANES, LANES)
      si[sl] = si[sl] + rep_base
    for k in range(CHUNK // LANES):
      sl = pl.ds(2 * CHUNK + k * LANES, LANES)
      si[sl] = si[sl] + (rep_base + POS_OFF)
    for k in range(CHUNK // LANES):
      sl = pl.ds(3 * CHUNK + k * LANES, LANES)
      si[sl] = si[sl] + (rep_base + NER_OFF)
    pltpu.async_copy(word_table.at[wi], wr, semg)
    pltpu.async_copy(small_table.at[si], sr, semg)

  def wait_gathers(slot):
    wi, si, wr, sr, _, semg, _ = slot
    pltpu.make_async_copy(word_table.at[wi], wr, semg).wait()
    pltpu.make_async_copy(small_table.at[si], sr, semg).wait()

  def scatter_ops(i, slot):
    wr, sr, sems = slot[2], slot[3], slot[6]
    base = wid * TOK_PER_WORKER + i * CHUNK
    tok = pl.ds(base, CHUNK)
    ops = [(wr, out.at[tok, pl.ds(0, EMB_DIM)], sems)]
    for f in range(4):
      ops.append((sr.at[pl.ds(f * CHUNK, CHUNK)],
                  out.at[tok, pl.ds(EMB_DIM + f * SMALL_DIM, SMALL_DIM)],
                  sems))
    return ops

  def fire_scatters(i, slot):
    for src, dst, sem in scatter_ops(i, slot):
      pltpu.async_copy(src, dst, sem)

  def drain_scatters(i, slot):
    for src, dst, sem in scatter_ops(i, slot):
      pltpu.make_async_copy(src, dst, sem).wait()

  start(0, slot_a)

  @pl.loop(0, N_PAIRS)
  def _pair(j):
    i0 = 2 * j
    i1 = i0 + 1

    @pl.when(j > 0)
    def _():
      drain_scatters(i0, slot_b)
    start(i1, slot_b)
    wait_gathers(slot_a)
    fire_scatters(i0, slot_a)

    @pl.when(j < N_PAIRS - 1)
    def _():
      drain_scatters(i0, slot_a)
      start(i0 + 2, slot_a)
    wait_gathers(slot_b)
    fire_scatters(i1, slot_b)

  drain_scatters(0, slot_a)
  drain_scatters(0, slot_b)


@jax.jit
def _run(words, subj, obj, pos, ner, word_table, small_table):
  mesh = plsc.VectorSubcoreMesh(
      core_axis_name="c", subcore_axis_name="s",
      num_cores=NUM_CORES, num_subcores=NUM_SUBCORES)
  grid_kernel = pl.kernel(
      _body,
      out_type=jax.ShapeDtypeStruct((N_TOK, OUT_DIM), jnp.float32),
      mesh=mesh,
      scratch_types=[
          pltpu.VMEM((CHUNK,), jnp.int32),
          pltpu.VMEM((4 * CHUNK,), jnp.int32),
          pltpu.VMEM((CHUNK, EMB_DIM), jnp.float32),
          pltpu.VMEM((4 * CHUNK, SMALL_DIM), jnp.float32),
          pltpu.VMEM((CHUNK,), jnp.int32),
          pltpu.VMEM((4 * CHUNK,), jnp.int32),
          pltpu.VMEM((CHUNK, EMB_DIM), jnp.float32),
          pltpu.VMEM((4 * CHUNK, SMALL_DIM), jnp.float32),
          pltpu.SemaphoreType.DMA,
          pltpu.SemaphoreType.DMA,
          pltpu.SemaphoreType.DMA,
          pltpu.SemaphoreType.DMA,
          pltpu.SemaphoreType.DMA,
          pltpu.SemaphoreType.DMA,
      ],
      compiler_params=pltpu.CompilerParams(use_tc_tiling_on_sc=False),
      name="embed_concat_sc",
  )
  return grid_kernel(words, subj, obj, pos, ner, word_table, small_table)


def kernel(words, pos, ner, subj_pos, obj_pos,
           word_table, pos_table, ner_table, position_table):
  small_table = jnp.tile(
      jnp.concatenate([position_table[MAXLEN:], pos_table, ner_table], axis=0),
      (NUM_WORKERS, 1))
  out = _run(words.reshape(N_TOK), subj_pos.reshape(N_TOK),
             obj_pos.reshape(N_TOK), pos.reshape(N_TOK), ner.reshape(N_TOK),
             word_table, small_table)
  return out.reshape(B, L, OUT_DIM)


# speedup vs baseline: 1.0464x; 1.0004x over previous
import jax
import jax.numpy as jnp
from jax import lax
from jax.experimental import pallas as pl
from jax.experimental.pallas import tpu as pltpu
from jax.experimental.pallas import tpu_sc as plsc

MAXLEN = 200
EMB_DIM = 64
SMALL_DIM = 32
OUT_DIM = EMB_DIM + 4 * SMALL_DIM

B, L = 4096, 200
N_TOK = B * L

POS_OFF = MAXLEN
NER_OFF = MAXLEN + 48
SMALL_ROWS = MAXLEN + 48 + 20

NUM_CORES = 2
NUM_SUBCORES = 16
NUM_WORKERS = NUM_CORES * NUM_SUBCORES
TOK_PER_WORKER = N_TOK // NUM_WORKERS
CHUNK = 320
N_CHUNKS = TOK_PER_WORKER // CHUNK
N_PAIRS = N_CHUNKS // 2
LANES = 16


def _body(words, subj, obj, pos, ner, word_table, small_table, out, *scratch):
  (wi_a, si_a, wr_a, sr_a, wi_b, si_b, wr_b, sr_b,
   semi_a, semg_a, sems_a, semi_b, semg_b, sems_b) = scratch
  slot_a = (wi_a, si_a, wr_a, sr_a, semi_a, semg_a, sems_a)
  slot_b = (wi_b, si_b, wr_b, sr_b, semi_b, semg_b, sems_b)

  c = lax.axis_index("c")
  s = lax.axis_index("s")
  wid = s * NUM_CORES + c
  rep_base = wid * SMALL_ROWS

  def idx_ops(i, slot):
    wi, si = slot[0], slot[1]
    semi = slot[4]
    base = wid * TOK_PER_WORKER + i * CHUNK
    tok = pl.ds(base, CHUNK)
    return [
        (words.at[tok], wi, semi),
        (subj.at[tok], si.at[pl.ds(0, CHUNK)], semi),
        (obj.at[tok], si.at[pl.ds(CHUNK, CHUNK)], semi),
        (pos.at[tok], si.at[pl.ds(2 * CHUNK, CHUNK)], semi),
        (ner.at[tok], si.at[pl.ds(3 * CHUNK, CHUNK)], semi),
    ]

  def start(i, slot):
    wi, si, wr, sr, semi, semg, _ = slot
    ops = idx_ops(i, slot)
    for src, dst, sem in ops:
      pltpu.async_copy(src, dst, sem)
    for src, dst, sem in ops:
      pltpu.make_async_copy(src, dst, sem).wait()
    for k in range(2 * CHUNK // LANES):
      sl = pl.ds(k * LANES, LANES)
      si[sl] = si[sl] + rep_base
    for k in range(CHUNK // LANES):
      sl = pl.ds(2 * CHUNK + k * LANES, LANES)
      si[sl] = si[sl] + (rep_base + POS_OFF)
    for k in range(CHUNK // LANES):
      sl = pl.ds(3 * CHUNK + k * LANES, LANES)
      si[sl] = si[sl] + (rep_base + NER_OFF)
    pltpu.async_copy(word_table.at[wi], wr, semg)
    pltpu.async_copy(small_table.at[si], sr, semg)

  def wait_gathers(slot):
    wi, si, wr, sr, _, semg, _ = slot
    pltpu.make_async_copy(word_table.at[wi], wr, semg).wait()
    pltpu.make_async_copy(small_table.at[si], sr, semg).wait()

  def scatter_ops(i, slot):
    wr, sr, sems = slot[2], slot[3], slot[6]
    base = wid * TOK_PER_WORKER + i * CHUNK
    tok = pl.ds(base, CHUNK)
    ops = [(wr, out.at[tok, pl.ds(0, EMB_DIM)], sems)]
    for f in range(4):
      ops.append((sr.at[pl.ds(f * CHUNK, CHUNK)],
                  out.at[tok, pl.ds(EMB_DIM + f * SMALL_DIM, SMALL_DIM)],
                  sems))
    return ops

  def fire_scatters(i, slot):
    for src, dst, sem in scatter_ops(i, slot):
      pltpu.async_copy(src, dst, sem)

  def drain_scatters(i, slot):
    for src, dst, sem in scatter_ops(i, slot):
      pltpu.make_async_copy(src, dst, sem).wait()

  start(0, slot_a)

  @pl.loop(0, N_PAIRS)
  def _pair(j):
    i0 = 2 * j
    i1 = i0 + 1

    @pl.when(j > 0)
    def _():
      drain_scatters(i0, slot_b)
    start(i1, slot_b)
    wait_gathers(slot_a)
    fire_scatters(i0, slot_a)

    @pl.when(j < N_PAIRS - 1)
    def _():
      drain_scatters(i0, slot_a)
      start(i0 + 2, slot_a)
    wait_gathers(slot_b)
    fire_scatters(i1, slot_b)

  drain_scatters(0, slot_a)
  drain_scatters(0, slot_b)


@jax.jit
def _run(words, subj, obj, pos, ner, word_table, small_table):
  mesh = plsc.VectorSubcoreMesh(
      core_axis_name="c", subcore_axis_name="s",
      num_cores=NUM_CORES, num_subcores=NUM_SUBCORES)
  grid_kernel = pl.kernel(
      _body,
      out_type=jax.ShapeDtypeStruct((N_TOK, OUT_DIM), jnp.float32),
      mesh=mesh,
      scratch_types=[
          pltpu.VMEM((CHUNK,), jnp.int32),
          pltpu.VMEM((4 * CHUNK,), jnp.int32),
          pltpu.VMEM((CHUNK, EMB_DIM), jnp.float32),
          pltpu.VMEM((4 * CHUNK, SMALL_DIM), jnp.float32),
          pltpu.VMEM((CHUNK,), jnp.int32),
          pltpu.VMEM((4 * CHUNK,), jnp.int32),
          pltpu.VMEM((CHUNK, EMB_DIM), jnp.float32),
          pltpu.VMEM((4 * CHUNK, SMALL_DIM), jnp.float32),
          pltpu.SemaphoreType.DMA,
          pltpu.SemaphoreType.DMA,
          pltpu.SemaphoreType.DMA,
          pltpu.SemaphoreType.DMA,
          pltpu.SemaphoreType.DMA,
          pltpu.SemaphoreType.DMA,
      ],
      compiler_params=pltpu.CompilerParams(use_tc_tiling_on_sc=False),
      name="embed_concat_sc",
  )
  return grid_kernel(words, subj, obj, pos, ner, word_table, small_table)


def kernel(words, pos, ner, subj_pos, obj_pos,
           word_table, pos_table, ner_table, position_table):
  small_table = jnp.tile(
      jnp.concatenate([position_table[MAXLEN:], pos_table, ner_table], axis=0),
      (NUM_WORKERS, 1))
  out = _run(words.reshape(N_TOK), subj_pos.reshape(N_TOK),
             obj_pos.reshape(N_TOK), pos.reshape(N_TOK), ner.reshape(N_TOK),
             word_table + 0.0, small_table)
  return out.reshape(B, L, OUT_DIM)


# speedup vs baseline: 1.0471x; 1.0007x over previous
import jax
import jax.numpy as jnp
from jax import lax
from jax.experimental import pallas as pl
from jax.experimental.pallas import tpu as pltpu
from jax.experimental.pallas import tpu_sc as plsc

MAXLEN = 200
EMB_DIM = 64
SMALL_DIM = 32
OUT_DIM = EMB_DIM + 4 * SMALL_DIM

B, L = 4096, 200
N_TOK = B * L

POS_OFF = MAXLEN
NER_OFF = MAXLEN + 48
SMALL_ROWS = MAXLEN + 48 + 20

NUM_CORES = 2
NUM_SUBCORES = 16
NUM_WORKERS = NUM_CORES * NUM_SUBCORES
TOK_PER_WORKER = N_TOK // NUM_WORKERS
CHUNK = 320
N_CHUNKS = TOK_PER_WORKER // CHUNK
N_PAIRS = N_CHUNKS // 2
LANES = 16


def _body(words, subj, obj, pos, ner, word_table, small_table, out, *scratch):
  (wi_a, si_a, wr_a, sr_a, wi_b, si_b, wr_b, sr_b,
   semi_a, semg_a, sems_a, semi_b, semg_b, sems_b) = scratch
  slot_a = (wi_a, si_a, wr_a, sr_a, semi_a, semg_a, sems_a)
  slot_b = (wi_b, si_b, wr_b, sr_b, semi_b, semg_b, sems_b)

  c = lax.axis_index("c")
  s = lax.axis_index("s")
  wid = s * NUM_CORES + c
  rep_base = wid * SMALL_ROWS

  def idx_ops(i, slot):
    wi, si = slot[0], slot[1]
    semi = slot[4]
    base = wid * TOK_PER_WORKER + i * CHUNK
    tok = pl.ds(base, CHUNK)
    return [
        (words.at[tok], wi, semi),
        (subj.at[tok], si.at[pl.ds(0, CHUNK)], semi),
        (obj.at[tok], si.at[pl.ds(CHUNK, CHUNK)], semi),
        (pos.at[tok], si.at[pl.ds(2 * CHUNK, CHUNK)], semi),
        (ner.at[tok], si.at[pl.ds(3 * CHUNK, CHUNK)], semi),
    ]

  def start(i, slot):
    wi, si, wr, sr, semi, semg, _ = slot
    ops = idx_ops(i, slot)
    for src, dst, sem in ops:
      pltpu.async_copy(src, dst, sem)
    for src, dst, sem in ops:
      pltpu.make_async_copy(src, dst, sem).wait()
    for k in range(2 * CHUNK // LANES):
      sl = pl.ds(k * LANES, LANES)
      si[sl] = si[sl] + rep_base
    for k in range(CHUNK // LANES):
      sl = pl.ds(2 * CHUNK + k * LANES, LANES)
      si[sl] = si[sl] + (rep_base + POS_OFF)
    for k in range(CHUNK // LANES):
      sl = pl.ds(3 * CHUNK + k * LANES, LANES)
      si[sl] = si[sl] + (rep_base + NER_OFF)
    pltpu.async_copy(word_table.at[wi], wr, semg)
    pltpu.async_copy(small_table.at[si], sr, semg)

  def wait_gathers(slot):
    wi, si, wr, sr, _, semg, _ = slot
    pltpu.make_async_copy(word_table.at[wi], wr, semg).wait()
    pltpu.make_async_copy(small_table.at[si], sr, semg).wait()

  def scatter_ops(i, slot):
    wr, sr, sems = slot[2], slot[3], slot[6]
    base = wid * TOK_PER_WORKER + i * CHUNK
    tok = pl.ds(base, CHUNK)
    ops = [(wr, out.at[tok, pl.ds(0, EMB_DIM)], sems)]
    for f in range(4):
      ops.append((sr.at[pl.ds(f * CHUNK, CHUNK)],
                  out.at[tok, pl.ds(EMB_DIM + f * SMALL_DIM, SMALL_DIM)],
                  sems))
    return ops

  def fire_scatters(i, slot):
    for src, dst, sem in scatter_ops(i, slot):
      pltpu.async_copy(src, dst, sem)

  def drain_scatters(i, slot):
    for src, dst, sem in scatter_ops(i, slot):
      pltpu.make_async_copy(src, dst, sem).wait()

  start(0, slot_a)

  @pl.loop(0, N_PAIRS)
  def _pair(j):
    i0 = 2 * j
    i1 = i0 + 1

    @pl.when(j > 0)
    def _():
      drain_scatters(i0, slot_b)
    start(i1, slot_b)
    wait_gathers(slot_a)
    fire_scatters(i0, slot_a)

    @pl.when(j < N_PAIRS - 1)
    def _():
      drain_scatters(i0, slot_a)
      start(i0 + 2, slot_a)
    wait_gathers(slot_b)
    fire_scatters(i1, slot_b)

  drain_scatters(0, slot_a)
  drain_scatters(0, slot_b)


@jax.jit
def _run(words, subj, obj, pos, ner, word_table, small_table):
  mesh = plsc.VectorSubcoreMesh(
      core_axis_name="c", subcore_axis_name="s",
      num_cores=NUM_CORES, num_subcores=NUM_SUBCORES)
  grid_kernel = pl.kernel(
      _body,
      out_type=jax.ShapeDtypeStruct((N_TOK, OUT_DIM), jnp.float32),
      mesh=mesh,
      scratch_types=[
          pltpu.VMEM((CHUNK,), jnp.int32),
          pltpu.VMEM((4 * CHUNK,), jnp.int32),
          pltpu.VMEM((CHUNK, EMB_DIM), jnp.float32),
          pltpu.VMEM((4 * CHUNK, SMALL_DIM), jnp.float32),
          pltpu.VMEM((CHUNK,), jnp.int32),
          pltpu.VMEM((4 * CHUNK,), jnp.int32),
          pltpu.VMEM((CHUNK, EMB_DIM), jnp.float32),
          pltpu.VMEM((4 * CHUNK, SMALL_DIM), jnp.float32),
          pltpu.SemaphoreType.DMA,
          pltpu.SemaphoreType.DMA,
          pltpu.SemaphoreType.DMA,
          pltpu.SemaphoreType.DMA,
          pltpu.SemaphoreType.DMA,
          pltpu.SemaphoreType.DMA,
      ],
      compiler_params=pltpu.CompilerParams(use_tc_tiling_on_sc=False),
      name="embed_concat_sc",
  )
  return grid_kernel(words, subj, obj, pos, ner, word_table, small_table)


def kernel(words, pos, ner, subj_pos, obj_pos,
           word_table, pos_table, ner_table, position_table):
  small_table = jnp.tile(
      jnp.concatenate([position_table[MAXLEN:], pos_table, ner_table], axis=0),
      (NUM_WORKERS, 1))
  out = _run(words.reshape(N_TOK), subj_pos.reshape(N_TOK),
             obj_pos.reshape(N_TOK), pos.reshape(N_TOK), ner.reshape(N_TOK),
             word_table, small_table)
  return out.reshape(B, L, OUT_DIM)
